# baseline (speedup 1.0000x reference)
import functools

import jax
import jax.numpy as jnp
from jax import lax
from jax.experimental import pallas as pl
from jax.experimental.pallas import tpu as pltpu
from jax.experimental.pallas import tpu_sc as plsc

NC = 2
NS = 16
LANES = 16
CH = 64


def _bf16_bits(v):
    u = lax.bitcast_convert_type(v, jnp.uint32)
    return (u + 0x7FFF + ((u >> 16) & 1)) >> 16


def _ea_body(a_ref, wlo_ref, whi_ref, blo_ref, bhi_ref, o_ref):
    a = a_ref[...]
    lo = jnp.dot(a, wlo_ref[...], preferred_element_type=jnp.float32
                 ) + blo_ref[...]
    hi = jnp.dot(a, whi_ref[...], preferred_element_type=jnp.float32
                 ) + bhi_ref[...]
    o_ref[...] = lax.bitcast_convert_type(
        _bf16_bits(lo) | (_bf16_bits(hi) << 16), jnp.int32)


def _edge_proj(er, Wblo, Wbhi, bblo, bbhi, e_pad):
    ER, EDP = er.shape
    W8 = Wblo.shape[1]
    BR = 1000
    grid = (ER // BR,)
    return pl.pallas_call(
        _ea_body,
        grid=grid,
        in_specs=[
            pl.BlockSpec((BR, EDP), lambda i: (i, 0)),
            pl.BlockSpec((EDP, W8), lambda i: (0, 0)),
            pl.BlockSpec((EDP, W8), lambda i: (0, 0)),
            pl.BlockSpec((1, W8), lambda i: (0, 0)),
            pl.BlockSpec((1, W8), lambda i: (0, 0)),
        ],
        out_specs=pl.BlockSpec((BR, W8), lambda i: (i, 0)),
        out_shape=jax.ShapeDtypeStruct((e_pad // 8, W8), jnp.int32),
    )(er, Wblo, Wbhi, bblo.reshape(1, W8), bbhi.reshape(1, W8))


def _sc_edge_body(n_nodes, ncht, src_hbm, dst_hbm, ea_hbm, x_hbm, z_hbm,
                  out_hbm, srcall, db0, db1, xb0, xb1, eb0, eb1,
                  aggr_sh, g0, g1, e0, e1, d0, d1, s0, s1):
    D = 128
    cid = lax.axis_index("c")
    sid = lax.axis_index("s")
    tid = cid * NS + sid
    xbs = (xb0, xb1)
    ebs = (eb0, eb1)
    dbs = (db0, db1)
    gs = (g0, g1)
    es = (e0, e1)
    ds = (d0, d1)
    ss = (s0, s1)

    RB = 640
    rem_z = n_nodes + 8 - (NS - 1) * RB
    rem_o = n_nodes - (NS - 1) * RB

    @pl.when(sid < NS - 1)
    def _():
        pltpu.sync_copy(z_hbm.at[pl.ds(sid * RB, RB)],
                        aggr_sh.at[pl.ds(sid * RB, RB)])

    @pl.when(sid == NS - 1)
    def _():
        pltpu.sync_copy(z_hbm.at[pl.ds((NS - 1) * RB, rem_z)],
                        aggr_sh.at[pl.ds((NS - 1) * RB, rem_z)])

    cbase = tid * ncht
    pltpu.sync_copy(src_hbm.at[pl.ds(cbase * CH, ncht * CH)], srcall)
    plsc.subcore_barrier()

    def issue(c, b):
        pltpu.async_copy(x_hbm.at[srcall.at[pl.ds(c * CH, CH)]], xbs[b],
                         gs[b])
        pltpu.async_copy(ea_hbm.at[pl.ds((cbase + c) * (CH // 8), CH // 8)],
                         ebs[b], es[b])
        pltpu.async_copy(dst_hbm.at[pl.ds((cbase + c) * CH, CH)], dbs[b],
                         ds[b])

    def wait_in(c, b):
        pltpu.make_async_copy(x_hbm.at[srcall.at[pl.ds(c * CH, CH)]],
                              xbs[b], gs[b]).wait()
        pltpu.make_async_copy(
            ea_hbm.at[pl.ds((cbase + c) * (CH // 8), CH // 8)],
            ebs[b], es[b]).wait()
        pltpu.make_async_copy(dst_hbm.at[pl.ds((cbase + c) * CH, CH)],
                              dbs[b], ds[b]).wait()

    mask_hi = jnp.full((LANES,), -65536, jnp.int32)
    shift16 = jnp.full((LANES,), 16, jnp.int32)

    def compute(b):
        xb, eb = xbs[b], ebs[b]

        def row_body(r, carry):
            for a in range(8):
                xr = 8 * r + a
                for k in range(D // 32):
                    ev = eb[r, pl.ds(a * 64 + k * LANES, LANES)]
                    elo = lax.bitcast_convert_type(
                        lax.shift_left(ev, shift16), jnp.float32)
                    ehi = lax.bitcast_convert_type(
                        lax.bitwise_and(ev, mask_hi), jnp.float32)
                    slo = pl.ds(k * 32, LANES)
                    shi = pl.ds(k * 32 + LANES, LANES)
                    xb[xr, slo] = jnp.maximum(xb[xr, slo] + elo, 0.0)
                    xb[xr, shi] = jnp.maximum(xb[xr, shi] + ehi, 0.0)
            return carry

        lax.fori_loop(0, CH // 8, row_body, 0)

    def scatter(b):
        pltpu.async_copy(xbs[b], aggr_sh.at[dbs[b]], ss[b], add=True)

    def wait_scatter(b):
        pltpu.make_async_copy(xbs[b], aggr_sh.at[dbs[b]], ss[b]).wait()

    issue(0, 0)

    def step(s, carry):
        c0 = s * 2

        @pl.when(s > 0)
        def _():
            wait_scatter(1)

        issue(c0 + 1, 1)
        wait_in(c0, 0)
        compute(0)
        scatter(0)

        wait_scatter(0)
        issue(c0 + 2, 0)
        wait_in(c0 + 1, 1)
        compute(1)
        scatter(1)
        return carry

    lax.fori_loop(0, (ncht - 1) // 2, step, 0)

    cl = ncht - 1
    wait_scatter(1)
    wait_in(cl, 0)
    compute(0)
    scatter(0)
    wait_scatter(0)
    plsc.subcore_barrier()

    @pl.when(sid < NS - 1)
    def _():
        pltpu.sync_copy(aggr_sh.at[pl.ds(sid * RB, RB)],
                        out_hbm.at[cid].at[pl.ds(sid * RB, RB)])

    @pl.when(sid == NS - 1)
    def _():
        pltpu.sync_copy(aggr_sh.at[pl.ds((NS - 1) * RB, rem_o)],
                        out_hbm.at[cid].at[pl.ds((NS - 1) * RB, rem_o)])


def _sc_edge_agg(src, dst, ea, x):
    N, D = x.shape
    ncht = src.shape[0] // (NC * NS * CH)
    assert (ncht - 1) % 2 == 0
    zeros = jnp.zeros((N + 8, D), jnp.float32)
    mesh = plsc.VectorSubcoreMesh(
        core_axis_name="c", subcore_axis_name="s", num_cores=NC,
        num_subcores=NS)
    f = pl.kernel(
        functools.partial(_sc_edge_body, N, ncht),
        out_type=jax.ShapeDtypeStruct((NC, N, D), jnp.float32),
        mesh=mesh,
        scratch_types=(
            [pltpu.VMEM((ncht * CH,), jnp.int32)]
            + [pltpu.VMEM((CH,), jnp.int32) for _ in range(2)]
            + [pltpu.VMEM((CH, D), jnp.float32) for _ in range(2)]
            + [pltpu.VMEM((CH // 8, 4 * D), jnp.int32) for _ in range(2)]
            + [pltpu.VMEM_SHARED((N + 8, D), jnp.float32)]
            + [pltpu.SemaphoreType.DMA for _ in range(8)]
        ),
    )
    return f(src, dst, ea, x, zeros)


def _mlp_body(x_ref, a0_ref, a1_ref, w1_ref, b1_ref, w2_ref, b2_ref,
              wg1_ref, bg1_ref, wg2_ref, bg2_ref, h_ref, g_ref):
    h0 = x_ref[...] + a0_ref[0] + a1_ref[0]
    t = jnp.maximum(
        jnp.dot(h0, w1_ref[...], preferred_element_type=jnp.float32)
        + b1_ref[...], 0.0)
    h = (jnp.dot(t, w2_ref[...], preferred_element_type=jnp.float32)
         + b2_ref[...])
    h_ref[...] = h
    gt = jnp.maximum(
        jnp.dot(h, wg1_ref[...], preferred_element_type=jnp.float32)
        + bg1_ref[...], 0.0)
    g_ref[...] = (jnp.dot(gt, wg2_ref[...], preferred_element_type=jnp.float32)
                  + bg2_ref[...])


def _node_mlp(x, aggr2, W1, b1, W2, b2, Wg1, bg1, Wg2, bg2):
    N, D = x.shape
    H = W1.shape[1]
    GD = Wg1.shape[1]
    BN = 2000
    grid = (N // BN,)
    return pl.pallas_call(
        _mlp_body,
        grid=grid,
        in_specs=[
            pl.BlockSpec((BN, D), lambda i: (i, 0)),
            pl.BlockSpec((1, BN, D), lambda i: (0, i, 0)),
            pl.BlockSpec((1, BN, D), lambda i: (1, i, 0)),
            pl.BlockSpec((D, H), lambda i: (0, 0)),
            pl.BlockSpec((1, H), lambda i: (0, 0)),
            pl.BlockSpec((H, H), lambda i: (0, 0)),
            pl.BlockSpec((1, H), lambda i: (0, 0)),
            pl.BlockSpec((H, GD), lambda i: (0, 0)),
            pl.BlockSpec((1, GD), lambda i: (0, 0)),
            pl.BlockSpec((GD, 1), lambda i: (0, 0)),
            pl.BlockSpec((1, 1), lambda i: (0, 0)),
        ],
        out_specs=[
            pl.BlockSpec((BN, H), lambda i: (i, 0)),
            pl.BlockSpec((BN, 1), lambda i: (i, 0)),
        ],
        out_shape=[
            jax.ShapeDtypeStruct((N, H), jnp.float32),
            jax.ShapeDtypeStruct((N, 1), jnp.float32),
        ],
    )(x, aggr2, aggr2, W1, b1.reshape(1, H), W2, b2.reshape(1, H),
      Wg1, bg1.reshape(1, GD), Wg2, bg2.reshape(1, 1))


def _pool_body(n_groups, h_ref, g_ref, b_ref, wh_ref, bh_ref,
               logits_ref, pooled_ref):
    g = g_ref[...]
    bid = b_ref[...]
    N = g.shape[0]
    gidx = lax.broadcasted_iota(jnp.int32, (N, n_groups), 1)
    mask = bid == gidx
    gmax = jnp.max(jnp.where(mask, g, -jnp.inf), axis=0, keepdims=True)
    gmax = jnp.where(gmax > -3e38, gmax, 0.0)
    gmax_n = jnp.sum(jnp.where(mask, gmax, 0.0), axis=1, keepdims=True)
    ge = jnp.exp(g - gmax_n)
    gsum = jnp.sum(jnp.where(mask, ge, 0.0), axis=0, keepdims=True)
    gsum_n = jnp.sum(jnp.where(mask, gsum, 0.0), axis=1, keepdims=True)
    alpha = ge / (gsum_n + 1e-16)
    wm = jnp.where(mask, alpha, 0.0)
    pooled = lax.dot_general(wm, h_ref[...], (((0,), (0,)), ((), ())),
                             preferred_element_type=jnp.float32)
    pooled_ref[...] = pooled
    logits_ref[...] = (
        jnp.dot(pooled, wh_ref[...], preferred_element_type=jnp.float32)
        + bh_ref[...])


def _pool_head(h, g, batch, Wh, bh, n_groups):
    N, H = h.shape
    C = Wh.shape[1]
    return pl.pallas_call(
        functools.partial(_pool_body, n_groups),
        out_shape=[
            jax.ShapeDtypeStruct((n_groups, C), jnp.float32),
            jax.ShapeDtypeStruct((n_groups, H), jnp.float32),
        ],
    )(h, g, batch.reshape(N, 1).astype(jnp.int32), Wh, bh.reshape(1, C))


def kernel(x, edge_index, edge_attr, batch, We, be, W1, b1, W2, b2,
           Wg1, bg1, Wg2, bg2, Wh, bh):
    N, D = x.shape
    E, ED = edge_attr.shape
    NW = NC * NS
    ncht = -(-E // (CH * NW))
    e_pad = ncht * NW * CH
    pad = e_pad - E
    src = jnp.concatenate([edge_index[0], jnp.zeros((pad,), jnp.int32)])
    dst = jnp.concatenate([edge_index[1], jnp.full((pad,), N, jnp.int32)])
    er = edge_attr.reshape(E // 8, 8 * ED)
    w = jnp.arange(D // 2)
    csel_lo = 32 * (w // 16) + w % 16
    csel_hi = csel_lo + 16
    eye8 = jnp.eye(8, dtype=jnp.float32)
    Wblo = jnp.einsum("ab,kw->akbw", eye8, We[:, csel_lo]).reshape(
        8 * ED, 8 * (D // 2))
    Wbhi = jnp.einsum("ab,kw->akbw", eye8, We[:, csel_hi]).reshape(
        8 * ED, 8 * (D // 2))
    bblo = jnp.tile(be[csel_lo], 8)
    bbhi = jnp.tile(be[csel_hi], 8)
    eap = _edge_proj(er, Wblo, Wbhi, bblo, bbhi, e_pad)
    aggr2 = _sc_edge_agg(src, dst, eap, x)
    h, g = _node_mlp(x, aggr2, W1, b1, W2, b2, Wg1, bg1, Wg2, bg2)
    logits, pooled = _pool_head(h, g, batch, Wh, bh, 64)
    return (logits, pooled)


# speedup vs baseline: 1.2389x; 1.2389x over previous
import functools

import jax
import jax.numpy as jnp
from jax import lax
from jax.experimental import pallas as pl
from jax.experimental.pallas import tpu as pltpu
from jax.experimental.pallas import tpu_sc as plsc

NC = 2
NS = 16
LANES = 16
CH = 80


def _bf16_bits(v):
    u = lax.bitcast_convert_type(v, jnp.uint32)
    return (u + 0x7FFF + ((u >> 16) & 1)) >> 16


def _ea_body(a1_ref, a2_ref, wlo_ref, whi_ref, blo_ref, bhi_ref, o_ref):
    def pack(a):
        lo = jnp.dot(a, wlo_ref[...], preferred_element_type=jnp.float32
                     ) + blo_ref[...]
        hi = jnp.dot(a, whi_ref[...], preferred_element_type=jnp.float32
                     ) + bhi_ref[...]
        return lax.bitcast_convert_type(
            _bf16_bits(lo) | (_bf16_bits(hi) << 16), jnp.int32)

    o_ref[...] = jnp.concatenate(
        [pack(a1_ref[...]), pack(a2_ref[...])], axis=1)


def _edge_proj(edge_attr, Wlo, Whi, blo, bhi):
    E, ED = edge_attr.shape
    DH = Wlo.shape[1]
    EH = E // 2
    BP = 2000
    grid = (EH // BP,)
    nb = EH // BP
    return pl.pallas_call(
        _ea_body,
        grid=grid,
        in_specs=[
            pl.BlockSpec((BP, ED), lambda i: (i, 0)),
            pl.BlockSpec((BP, ED), lambda i, _nb=nb: (i + _nb, 0)),
            pl.BlockSpec((ED, DH), lambda i: (0, 0)),
            pl.BlockSpec((ED, DH), lambda i: (0, 0)),
            pl.BlockSpec((1, DH), lambda i: (0, 0)),
            pl.BlockSpec((1, DH), lambda i: (0, 0)),
        ],
        out_specs=pl.BlockSpec((BP, 2 * DH), lambda i: (i, 0)),
        out_shape=jax.ShapeDtypeStruct((EH, 2 * DH), jnp.int32),
    )(edge_attr, edge_attr, Wlo, Whi,
      blo.reshape(1, DH), bhi.reshape(1, DH))


def _sc_edge_body(n_nodes, ncht, src_hbm, dst_hbm, ea_hbm, x_hbm, z_hbm,
                  out_hbm, srcall, db0, db1, xb0, xb1, eb0, eb1,
                  aggr_sh, g0, g1, e0, e1, d0, d1, s0, s1):
    D = 128
    cid = lax.axis_index("c")
    sid = lax.axis_index("s")
    tid = cid * NS + sid
    xbs = (xb0, xb1)
    ebs = (eb0, eb1)
    dbs = (db0, db1)
    gs = (g0, g1)
    es = (e0, e1)
    ds = (d0, d1)
    ss = (s0, s1)

    RB = 640
    rem_z = n_nodes - (NS - 1) * RB
    rem_o = rem_z

    @pl.when(sid < NS - 1)
    def _():
        pltpu.sync_copy(z_hbm.at[pl.ds(sid * RB, RB)],
                        aggr_sh.at[pl.ds(sid * RB, RB)])

    @pl.when(sid == NS - 1)
    def _():
        pltpu.sync_copy(z_hbm.at[pl.ds((NS - 1) * RB, rem_z)],
                        aggr_sh.at[pl.ds((NS - 1) * RB, rem_z)])

    cbase = tid * ncht
    pltpu.sync_copy(src_hbm.at[pl.ds(cbase * CH, ncht * CH)], srcall)
    plsc.subcore_barrier()

    def issue(c, b):
        pltpu.async_copy(x_hbm.at[srcall.at[pl.ds(c * CH, CH)]], xbs[b],
                         gs[b])
        pltpu.async_copy(ea_hbm.at[pl.ds((cbase + c) * (CH // 2), CH // 2)],
                         ebs[b], es[b])
        pltpu.async_copy(dst_hbm.at[pl.ds((cbase + c) * CH, CH)], dbs[b],
                         ds[b])

    def wait_in(c, b):
        pltpu.make_async_copy(x_hbm.at[srcall.at[pl.ds(c * CH, CH)]],
                              xbs[b], gs[b]).wait()
        pltpu.make_async_copy(
            ea_hbm.at[pl.ds((cbase + c) * (CH // 2), CH // 2)],
            ebs[b], es[b]).wait()
        pltpu.make_async_copy(dst_hbm.at[pl.ds((cbase + c) * CH, CH)],
                              dbs[b], ds[b]).wait()

    mask_hi = jnp.full((LANES,), -65536, jnp.int32)
    shift16 = jnp.full((LANES,), 16, jnp.int32)

    def compute(b):
        xb, eb = xbs[b], ebs[b]

        def row_body(r, carry):
            for half in range(2):
                xr = half * (CH // 2) + r
                for k in range(D // 32):
                    ev = eb[r, pl.ds(half * (D // 2) + k * LANES, LANES)]
                    elo = lax.bitcast_convert_type(
                        lax.shift_left(ev, shift16), jnp.float32)
                    ehi = lax.bitcast_convert_type(
                        lax.bitwise_and(ev, mask_hi), jnp.float32)
                    slo = pl.ds(k * 32, LANES)
                    shi = pl.ds(k * 32 + LANES, LANES)
                    xb[xr, slo] = jnp.maximum(xb[xr, slo] + elo, 0.0)
                    xb[xr, shi] = jnp.maximum(xb[xr, shi] + ehi, 0.0)
            return carry

        lax.fori_loop(0, CH // 2, row_body, 0)

    def scatter(b):
        pltpu.async_copy(xbs[b], aggr_sh.at[dbs[b]], ss[b], add=True)

    def wait_scatter(b):
        pltpu.make_async_copy(xbs[b], aggr_sh.at[dbs[b]], ss[b]).wait()

    issue(0, 0)

    def step(s, carry):
        c0 = s * 2

        @pl.when(s > 0)
        def _():
            wait_scatter(1)

        issue(c0 + 1, 1)
        wait_in(c0, 0)
        compute(0)
        scatter(0)

        wait_scatter(0)
        issue(c0 + 2, 0)
        wait_in(c0 + 1, 1)
        compute(1)
        scatter(1)
        return carry

    lax.fori_loop(0, (ncht - 1) // 2, step, 0)

    cl = ncht - 1
    wait_scatter(1)
    wait_in(cl, 0)
    compute(0)
    scatter(0)
    wait_scatter(0)
    plsc.subcore_barrier()

    @pl.when(sid < NS - 1)
    def _():
        pltpu.sync_copy(aggr_sh.at[pl.ds(sid * RB, RB)],
                        out_hbm.at[cid].at[pl.ds(sid * RB, RB)])

    @pl.when(sid == NS - 1)
    def _():
        pltpu.sync_copy(aggr_sh.at[pl.ds((NS - 1) * RB, rem_o)],
                        out_hbm.at[cid].at[pl.ds((NS - 1) * RB, rem_o)])


def _sc_edge_agg(src, dst, ea, x):
    N, D = x.shape
    ncht = src.shape[0] // (NC * NS * CH)
    assert (ncht - 1) % 2 == 0
    zeros = jnp.zeros((N, D), jnp.float32)
    mesh = plsc.VectorSubcoreMesh(
        core_axis_name="c", subcore_axis_name="s", num_cores=NC,
        num_subcores=NS)
    f = pl.kernel(
        functools.partial(_sc_edge_body, N, ncht),
        out_type=jax.ShapeDtypeStruct((NC, N, D), jnp.float32),
        mesh=mesh,
        scratch_types=(
            [pltpu.VMEM((ncht * CH,), jnp.int32)]
            + [pltpu.VMEM((CH,), jnp.int32) for _ in range(2)]
            + [pltpu.VMEM((CH, D), jnp.float32) for _ in range(2)]
            + [pltpu.VMEM((CH // 2, D), jnp.int32) for _ in range(2)]
            + [pltpu.VMEM_SHARED((N, D), jnp.float32)]
            + [pltpu.SemaphoreType.DMA for _ in range(8)]
        ),
    )
    return f(src, dst, ea, x, zeros)


def _mlp_body(x_ref, a0_ref, a1_ref, w1_ref, b1_ref, w2_ref, b2_ref,
              wg1_ref, bg1_ref, wg2_ref, bg2_ref, h_ref, g_ref):
    h0 = x_ref[...] + a0_ref[0] + a1_ref[0]
    t = jnp.maximum(
        jnp.dot(h0, w1_ref[...], preferred_element_type=jnp.float32)
        + b1_ref[...], 0.0)
    h = (jnp.dot(t, w2_ref[...], preferred_element_type=jnp.float32)
         + b2_ref[...])
    h_ref[...] = h
    gt = jnp.maximum(
        jnp.dot(h, wg1_ref[...], preferred_element_type=jnp.float32)
        + bg1_ref[...], 0.0)
    g_ref[...] = (jnp.dot(gt, wg2_ref[...], preferred_element_type=jnp.float32)
                  + bg2_ref[...])


def _node_mlp(x, aggr2, W1, b1, W2, b2, Wg1, bg1, Wg2, bg2):
    N, D = x.shape
    H = W1.shape[1]
    GD = Wg1.shape[1]
    BN = 2000
    grid = (N // BN,)
    return pl.pallas_call(
        _mlp_body,
        grid=grid,
        in_specs=[
            pl.BlockSpec((BN, D), lambda i: (i, 0)),
            pl.BlockSpec((1, BN, D), lambda i: (0, i, 0)),
            pl.BlockSpec((1, BN, D), lambda i: (1, i, 0)),
            pl.BlockSpec((D, H), lambda i: (0, 0)),
            pl.BlockSpec((1, H), lambda i: (0, 0)),
            pl.BlockSpec((H, H), lambda i: (0, 0)),
            pl.BlockSpec((1, H), lambda i: (0, 0)),
            pl.BlockSpec((H, GD), lambda i: (0, 0)),
            pl.BlockSpec((1, GD), lambda i: (0, 0)),
            pl.BlockSpec((GD, 1), lambda i: (0, 0)),
            pl.BlockSpec((1, 1), lambda i: (0, 0)),
        ],
        out_specs=[
            pl.BlockSpec((BN, H), lambda i: (i, 0)),
            pl.BlockSpec((BN, 1), lambda i: (i, 0)),
        ],
        out_shape=[
            jax.ShapeDtypeStruct((N, H), jnp.float32),
            jax.ShapeDtypeStruct((N, 1), jnp.float32),
        ],
    )(x, aggr2, aggr2, W1, b1.reshape(1, H), W2, b2.reshape(1, H),
      Wg1, bg1.reshape(1, GD), Wg2, bg2.reshape(1, 1))


def _pool_body(n_groups, h_ref, g_ref, b_ref, wh_ref, bh_ref,
               logits_ref, pooled_ref):
    g = g_ref[...]
    bid = b_ref[...]
    N = g.shape[0]
    gidx = lax.broadcasted_iota(jnp.int32, (N, n_groups), 1)
    mask = bid == gidx
    gmax = jnp.max(jnp.where(mask, g, -jnp.inf), axis=0, keepdims=True)
    gmax = jnp.where(gmax > -3e38, gmax, 0.0)
    gmax_n = jnp.sum(jnp.where(mask, gmax, 0.0), axis=1, keepdims=True)
    ge = jnp.exp(g - gmax_n)
    gsum = jnp.sum(jnp.where(mask, ge, 0.0), axis=0, keepdims=True)
    gsum_n = jnp.sum(jnp.where(mask, gsum, 0.0), axis=1, keepdims=True)
    alpha = ge / (gsum_n + 1e-16)
    wm = jnp.where(mask, alpha, 0.0)
    pooled = lax.dot_general(wm, h_ref[...], (((0,), (0,)), ((), ())),
                             preferred_element_type=jnp.float32)
    pooled_ref[...] = pooled
    logits_ref[...] = (
        jnp.dot(pooled, wh_ref[...], preferred_element_type=jnp.float32)
        + bh_ref[...])


def _pool_head(h, g, batch, Wh, bh, n_groups):
    N, H = h.shape
    C = Wh.shape[1]
    return pl.pallas_call(
        functools.partial(_pool_body, n_groups),
        out_shape=[
            jax.ShapeDtypeStruct((n_groups, C), jnp.float32),
            jax.ShapeDtypeStruct((n_groups, H), jnp.float32),
        ],
    )(h, g, batch.reshape(N, 1).astype(jnp.int32), Wh, bh.reshape(1, C))


def kernel(x, edge_index, edge_attr, batch, We, be, W1, b1, W2, b2,
           Wg1, bg1, Wg2, bg2, Wh, bh):
    N, D = x.shape
    E, ED = edge_attr.shape
    NW = NC * NS
    EH = E // 2
    CR = CH // 2
    assert E % (CH * NW) == 0
    def reorder(v):
        return jnp.concatenate(
            [v[:EH].reshape(EH // CR, CR), v[EH:].reshape(EH // CR, CR)],
            axis=1).reshape(E)

    src = reorder(edge_index[0])
    dst = reorder(edge_index[1])
    w = jnp.arange(D // 2)
    csel_lo = 32 * (w // 16) + w % 16
    csel_hi = csel_lo + 16
    eap = _edge_proj(edge_attr, We[:, csel_lo], We[:, csel_hi],
                     be[csel_lo], be[csel_hi])
    aggr2 = _sc_edge_agg(src, dst, eap, x)
    h, g = _node_mlp(x, aggr2, W1, b1, W2, b2, Wg1, bg1, Wg2, bg2)
    logits, pooled = _pool_head(h, g, batch, Wh, bh, 64)
    return (logits, pooled)


# speedup vs baseline: 1.3080x; 1.0558x over previous
import functools

import jax
import jax.numpy as jnp
from jax import lax
from jax.experimental import pallas as pl
from jax.experimental.pallas import tpu as pltpu
from jax.experimental.pallas import tpu_sc as plsc

NC = 2
NS = 16
LANES = 16
CH = 80


def _bf16_bits(v):
    u = lax.bitcast_convert_type(v, jnp.uint32)
    return (u + 0x7FFF + ((u >> 16) & 1)) >> 16


def _ea_body(a1_ref, a2_ref, wlo_ref, whi_ref, blo_ref, bhi_ref, o_ref):
    def pack(a):
        lo = jnp.dot(a, wlo_ref[...], preferred_element_type=jnp.float32
                     ) + blo_ref[...]
        hi = jnp.dot(a, whi_ref[...], preferred_element_type=jnp.float32
                     ) + bhi_ref[...]
        return lax.bitcast_convert_type(
            _bf16_bits(lo) | (_bf16_bits(hi) << 16), jnp.int32)

    o_ref[...] = jnp.concatenate(
        [pack(a1_ref[...]), pack(a2_ref[...])], axis=1)


def _edge_proj(edge_attr, Wlo, Whi, blo, bhi):
    E, ED = edge_attr.shape
    DH = Wlo.shape[1]
    EH = E // 2
    BP = 4000
    grid = (EH // BP,)
    nb = EH // BP
    return pl.pallas_call(
        _ea_body,
        grid=grid,
        in_specs=[
            pl.BlockSpec((BP, ED), lambda i: (i, 0)),
            pl.BlockSpec((BP, ED), lambda i, _nb=nb: (i + _nb, 0)),
            pl.BlockSpec((ED, DH), lambda i: (0, 0)),
            pl.BlockSpec((ED, DH), lambda i: (0, 0)),
            pl.BlockSpec((1, DH), lambda i: (0, 0)),
            pl.BlockSpec((1, DH), lambda i: (0, 0)),
        ],
        out_specs=pl.BlockSpec((BP, 2 * DH), lambda i: (i, 0)),
        out_shape=jax.ShapeDtypeStruct((EH, 2 * DH), jnp.int32),
    )(edge_attr, edge_attr, Wlo, Whi,
      blo.reshape(1, DH), bhi.reshape(1, DH))


def _sc_edge_body(n_nodes, ncht, src_hbm, dst_hbm, ea_hbm, x_hbm, z_hbm,
                  out_hbm, srcall, db0, db1, xb0, xb1, eb0, eb1,
                  aggr_sh, g0, g1, e0, e1, d0, d1, s0, s1):
    D = 128
    cid = lax.axis_index("c")
    sid = lax.axis_index("s")
    tid = cid * NS + sid
    xbs = (xb0, xb1)
    ebs = (eb0, eb1)
    dbs = (db0, db1)
    gs = (g0, g1)
    es = (e0, e1)
    ds = (d0, d1)
    ss = (s0, s1)

    RB = 640
    rem_z = n_nodes - (NS - 1) * RB
    rem_o = rem_z

    @pl.when(sid < NS - 1)
    def _():
        pltpu.sync_copy(z_hbm.at[pl.ds(sid * RB, RB)],
                        aggr_sh.at[pl.ds(sid * RB, RB)])

    @pl.when(sid == NS - 1)
    def _():
        pltpu.sync_copy(z_hbm.at[pl.ds((NS - 1) * RB, rem_z)],
                        aggr_sh.at[pl.ds((NS - 1) * RB, rem_z)])

    cbase = tid * ncht
    pltpu.sync_copy(src_hbm.at[pl.ds(cbase * CH, ncht * CH)], srcall)
    plsc.subcore_barrier()

    def issue(c, b):
        pltpu.async_copy(x_hbm.at[srcall.at[pl.ds(c * CH, CH)]], xbs[b],
                         gs[b])
        pltpu.async_copy(ea_hbm.at[pl.ds((cbase + c) * (CH // 2), CH // 2)],
                         ebs[b], es[b])
        pltpu.async_copy(dst_hbm.at[pl.ds((cbase + c) * CH, CH)], dbs[b],
                         ds[b])

    def wait_in(c, b):
        pltpu.make_async_copy(x_hbm.at[srcall.at[pl.ds(c * CH, CH)]],
                              xbs[b], gs[b]).wait()
        pltpu.make_async_copy(
            ea_hbm.at[pl.ds((cbase + c) * (CH // 2), CH // 2)],
            ebs[b], es[b]).wait()
        pltpu.make_async_copy(dst_hbm.at[pl.ds((cbase + c) * CH, CH)],
                              dbs[b], ds[b]).wait()

    mask_hi = jnp.full((LANES,), -65536, jnp.int32)
    shift16 = jnp.full((LANES,), 16, jnp.int32)

    def compute(b):
        xb, eb = xbs[b], ebs[b]

        def row_body(r, carry):
            for half in range(2):
                xr = half * (CH // 2) + r
                for k in range(D // 32):
                    ev = eb[r, pl.ds(half * (D // 2) + k * LANES, LANES)]
                    elo = lax.bitcast_convert_type(
                        lax.shift_left(ev, shift16), jnp.float32)
                    ehi = lax.bitcast_convert_type(
                        lax.bitwise_and(ev, mask_hi), jnp.float32)
                    slo = pl.ds(k * 32, LANES)
                    shi = pl.ds(k * 32 + LANES, LANES)
                    xb[xr, slo] = jnp.maximum(xb[xr, slo] + elo, 0.0)
                    xb[xr, shi] = jnp.maximum(xb[xr, shi] + ehi, 0.0)
            return carry

        lax.fori_loop(0, CH // 2, row_body, 0)

    def scatter(b):
        pltpu.async_copy(xbs[b], aggr_sh.at[dbs[b]], ss[b], add=True)

    def wait_scatter(b):
        pltpu.make_async_copy(xbs[b], aggr_sh.at[dbs[b]], ss[b]).wait()

    issue(0, 0)

    def step(s, carry):
        c0 = s * 2

        @pl.when(s > 0)
        def _():
            wait_scatter(1)

        issue(c0 + 1, 1)
        wait_in(c0, 0)
        compute(0)
        scatter(0)

        wait_scatter(0)
        issue(c0 + 2, 0)
        wait_in(c0 + 1, 1)
        compute(1)
        scatter(1)
        return carry

    lax.fori_loop(0, (ncht - 1) // 2, step, 0)

    cl = ncht - 1
    wait_scatter(1)
    wait_in(cl, 0)
    compute(0)
    scatter(0)
    wait_scatter(0)
    plsc.subcore_barrier()

    @pl.when(sid < NS - 1)
    def _():
        pltpu.sync_copy(aggr_sh.at[pl.ds(sid * RB, RB)],
                        out_hbm.at[cid].at[pl.ds(sid * RB, RB)])

    @pl.when(sid == NS - 1)
    def _():
        pltpu.sync_copy(aggr_sh.at[pl.ds((NS - 1) * RB, rem_o)],
                        out_hbm.at[cid].at[pl.ds((NS - 1) * RB, rem_o)])


def _sc_edge_agg(src, dst, ea, x):
    N, D = x.shape
    ncht = src.shape[0] // (NC * NS * CH)
    assert (ncht - 1) % 2 == 0
    zeros = jnp.zeros((N, D), jnp.float32)
    mesh = plsc.VectorSubcoreMesh(
        core_axis_name="c", subcore_axis_name="s", num_cores=NC,
        num_subcores=NS)
    f = pl.kernel(
        functools.partial(_sc_edge_body, N, ncht),
        out_type=jax.ShapeDtypeStruct((NC, N, D), jnp.float32),
        mesh=mesh,
        scratch_types=(
            [pltpu.VMEM((ncht * CH,), jnp.int32)]
            + [pltpu.VMEM((CH,), jnp.int32) for _ in range(2)]
            + [pltpu.VMEM((CH, D), jnp.float32) for _ in range(2)]
            + [pltpu.VMEM((CH // 2, D), jnp.int32) for _ in range(2)]
            + [pltpu.VMEM_SHARED((N, D), jnp.float32)]
            + [pltpu.SemaphoreType.DMA for _ in range(8)]
        ),
    )
    return f(src, dst, ea, x, zeros)


def _mlp_body(x_ref, a0_ref, a1_ref, w1_ref, b1_ref, w2_ref, b2_ref,
              wg1_ref, bg1_ref, wg2_ref, bg2_ref, h_ref, g_ref):
    h0 = x_ref[...] + a0_ref[0] + a1_ref[0]
    t = jnp.maximum(
        jnp.dot(h0, w1_ref[...], preferred_element_type=jnp.float32)
        + b1_ref[...], 0.0)
    h = (jnp.dot(t, w2_ref[...], preferred_element_type=jnp.float32)
         + b2_ref[...])
    h_ref[...] = h
    gt = jnp.maximum(
        jnp.dot(h, wg1_ref[...], preferred_element_type=jnp.float32)
        + bg1_ref[...], 0.0)
    g_ref[...] = (jnp.dot(gt, wg2_ref[...], preferred_element_type=jnp.float32)
                  + bg2_ref[...])


def _node_mlp(x, aggr2, W1, b1, W2, b2, Wg1, bg1, Wg2, bg2):
    N, D = x.shape
    H = W1.shape[1]
    GD = Wg1.shape[1]
    BN = 2000
    grid = (N // BN,)
    return pl.pallas_call(
        _mlp_body,
        grid=grid,
        in_specs=[
            pl.BlockSpec((BN, D), lambda i: (i, 0)),
            pl.BlockSpec((1, BN, D), lambda i: (0, i, 0)),
            pl.BlockSpec((1, BN, D), lambda i: (1, i, 0)),
            pl.BlockSpec((D, H), lambda i: (0, 0)),
            pl.BlockSpec((1, H), lambda i: (0, 0)),
            pl.BlockSpec((H, H), lambda i: (0, 0)),
            pl.BlockSpec((1, H), lambda i: (0, 0)),
            pl.BlockSpec((H, GD), lambda i: (0, 0)),
            pl.BlockSpec((1, GD), lambda i: (0, 0)),
            pl.BlockSpec((GD, 1), lambda i: (0, 0)),
            pl.BlockSpec((1, 1), lambda i: (0, 0)),
        ],
        out_specs=[
            pl.BlockSpec((BN, H), lambda i: (i, 0)),
            pl.BlockSpec((BN, 1), lambda i: (i, 0)),
        ],
        out_shape=[
            jax.ShapeDtypeStruct((N, H), jnp.float32),
            jax.ShapeDtypeStruct((N, 1), jnp.float32),
        ],
    )(x, aggr2, aggr2, W1, b1.reshape(1, H), W2, b2.reshape(1, H),
      Wg1, bg1.reshape(1, GD), Wg2, bg2.reshape(1, 1))


def _pool_body(n_groups, h_ref, g_ref, b_ref, wh_ref, bh_ref,
               logits_ref, pooled_ref):
    g = g_ref[...]
    bid = b_ref[...]
    N = g.shape[0]
    gidx = lax.broadcasted_iota(jnp.int32, (N, n_groups), 1)
    mask = bid == gidx
    gmax = jnp.max(jnp.where(mask, g, -jnp.inf), axis=0, keepdims=True)
    gmax = jnp.where(gmax > -3e38, gmax, 0.0)
    gmax_n = jnp.sum(jnp.where(mask, gmax, 0.0), axis=1, keepdims=True)
    ge = jnp.exp(g - gmax_n)
    gsum = jnp.sum(jnp.where(mask, ge, 0.0), axis=0, keepdims=True)
    gsum_n = jnp.sum(jnp.where(mask, gsum, 0.0), axis=1, keepdims=True)
    alpha = ge / (gsum_n + 1e-16)
    wm = jnp.where(mask, alpha, 0.0)
    pooled = lax.dot_general(wm, h_ref[...], (((0,), (0,)), ((), ())),
                             preferred_element_type=jnp.float32)
    pooled_ref[...] = pooled
    logits_ref[...] = (
        jnp.dot(pooled, wh_ref[...], preferred_element_type=jnp.float32)
        + bh_ref[...])


def _pool_head(h, g, batch, Wh, bh, n_groups):
    N, H = h.shape
    C = Wh.shape[1]
    return pl.pallas_call(
        functools.partial(_pool_body, n_groups),
        out_shape=[
            jax.ShapeDtypeStruct((n_groups, C), jnp.float32),
            jax.ShapeDtypeStruct((n_groups, H), jnp.float32),
        ],
    )(h, g, batch.reshape(N, 1).astype(jnp.int32), Wh, bh.reshape(1, C))


def kernel(x, edge_index, edge_attr, batch, We, be, W1, b1, W2, b2,
           Wg1, bg1, Wg2, bg2, Wh, bh):
    N, D = x.shape
    E, ED = edge_attr.shape
    NW = NC * NS
    EH = E // 2
    CR = CH // 2
    assert E % (CH * NW) == 0
    def reorder(v):
        return jnp.concatenate(
            [v[:EH].reshape(EH // CR, CR), v[EH:].reshape(EH // CR, CR)],
            axis=1).reshape(E)

    src = reorder(edge_index[0])
    dst = reorder(edge_index[1])
    w = jnp.arange(D // 2)
    csel_lo = 32 * (w // 16) + w % 16
    csel_hi = csel_lo + 16
    eap = _edge_proj(edge_attr, We[:, csel_lo], We[:, csel_hi],
                     be[csel_lo], be[csel_hi])
    aggr2 = _sc_edge_agg(src, dst, eap, x)
    h, g = _node_mlp(x, aggr2, W1, b1, W2, b2, Wg1, bg1, Wg2, bg2)
    logits, pooled = _pool_head(h, g, batch, Wh, bh, 64)
    return (logits, pooled)


# speedup vs baseline: 1.3388x; 1.0235x over previous
import functools

import jax
import jax.numpy as jnp
from jax import lax
from jax.experimental import pallas as pl
from jax.experimental.pallas import tpu as pltpu
from jax.experimental.pallas import tpu_sc as plsc

NC = 2
NS = 16
LANES = 16
CH = 80


def _bf16_bits(v):
    u = lax.bitcast_convert_type(v, jnp.uint32)
    return (u + 0x7FFF + ((u >> 16) & 1)) >> 16


def _ea_body(a1_ref, a2_ref, wlo_ref, whi_ref, blo_ref, bhi_ref, o_ref):
    def pack(a):
        lo = jnp.dot(a, wlo_ref[...], preferred_element_type=jnp.float32
                     ) + blo_ref[...]
        hi = jnp.dot(a, whi_ref[...], preferred_element_type=jnp.float32
                     ) + bhi_ref[...]
        return lax.bitcast_convert_type(
            _bf16_bits(lo) | (_bf16_bits(hi) << 16), jnp.int32)

    o_ref[...] = jnp.concatenate(
        [pack(a1_ref[...]), pack(a2_ref[...])], axis=1)


def _edge_proj(edge_attr, Wlo, Whi, blo, bhi):
    E, ED = edge_attr.shape
    DH = Wlo.shape[1]
    EH = E // 2
    BP = 8000
    grid = (EH // BP,)
    nb = EH // BP
    return pl.pallas_call(
        _ea_body,
        grid=grid,
        in_specs=[
            pl.BlockSpec((BP, ED), lambda i: (i, 0)),
            pl.BlockSpec((BP, ED), lambda i, _nb=nb: (i + _nb, 0)),
            pl.BlockSpec((ED, DH), lambda i: (0, 0)),
            pl.BlockSpec((ED, DH), lambda i: (0, 0)),
            pl.BlockSpec((1, DH), lambda i: (0, 0)),
            pl.BlockSpec((1, DH), lambda i: (0, 0)),
        ],
        out_specs=pl.BlockSpec((BP, 2 * DH), lambda i: (i, 0)),
        out_shape=jax.ShapeDtypeStruct((EH, 2 * DH), jnp.int32),
    )(edge_attr, edge_attr, Wlo, Whi,
      blo.reshape(1, DH), bhi.reshape(1, DH))


def _sc_edge_body(n_nodes, ncht, src_hbm, dst_hbm, ea_hbm, x_hbm, z_hbm,
                  out_hbm, srcall, db0, db1, xb0, xb1, eb0, eb1,
                  aggr_sh, g0, g1, e0, e1, d0, d1, s0, s1):
    D = 128
    cid = lax.axis_index("c")
    sid = lax.axis_index("s")
    tid = cid * NS + sid
    xbs = (xb0, xb1)
    ebs = (eb0, eb1)
    dbs = (db0, db1)
    gs = (g0, g1)
    es = (e0, e1)
    ds = (d0, d1)
    ss = (s0, s1)

    RB = 640
    rem_z = n_nodes - (NS - 1) * RB
    rem_o = rem_z

    @pl.when(sid < NS - 1)
    def _():
        pltpu.sync_copy(z_hbm.at[pl.ds(sid * RB, RB)],
                        aggr_sh.at[pl.ds(sid * RB, RB)])

    @pl.when(sid == NS - 1)
    def _():
        pltpu.sync_copy(z_hbm.at[pl.ds((NS - 1) * RB, rem_z)],
                        aggr_sh.at[pl.ds((NS - 1) * RB, rem_z)])

    cbase = tid * ncht
    pltpu.sync_copy(src_hbm.at[pl.ds(cbase * CH, ncht * CH)], srcall)
    plsc.subcore_barrier()

    def issue(c, b):
        pltpu.async_copy(x_hbm.at[srcall.at[pl.ds(c * CH, CH)]], xbs[b],
                         gs[b])
        pltpu.async_copy(ea_hbm.at[pl.ds((cbase + c) * (CH // 2), CH // 2)],
                         ebs[b], es[b])
        pltpu.async_copy(dst_hbm.at[pl.ds((cbase + c) * CH, CH)], dbs[b],
                         ds[b])

    def wait_in(c, b):
        pltpu.make_async_copy(x_hbm.at[srcall.at[pl.ds(c * CH, CH)]],
                              xbs[b], gs[b]).wait()
        pltpu.make_async_copy(
            ea_hbm.at[pl.ds((cbase + c) * (CH // 2), CH // 2)],
            ebs[b], es[b]).wait()
        pltpu.make_async_copy(dst_hbm.at[pl.ds((cbase + c) * CH, CH)],
                              dbs[b], ds[b]).wait()

    mask_hi = jnp.full((LANES,), -65536, jnp.int32)
    shift16 = jnp.full((LANES,), 16, jnp.int32)

    def compute(b):
        xb, eb = xbs[b], ebs[b]

        def row_body(r, carry):
            for half in range(2):
                xr = half * (CH // 2) + r
                for k in range(D // 32):
                    ev = eb[r, pl.ds(half * (D // 2) + k * LANES, LANES)]
                    elo = lax.bitcast_convert_type(
                        lax.shift_left(ev, shift16), jnp.float32)
                    ehi = lax.bitcast_convert_type(
                        lax.bitwise_and(ev, mask_hi), jnp.float32)
                    slo = pl.ds(k * 32, LANES)
                    shi = pl.ds(k * 32 + LANES, LANES)
                    xb[xr, slo] = jnp.maximum(xb[xr, slo] + elo, 0.0)
                    xb[xr, shi] = jnp.maximum(xb[xr, shi] + ehi, 0.0)
            return carry

        lax.fori_loop(0, CH // 2, row_body, 0)

    def scatter(b):
        pltpu.async_copy(xbs[b], aggr_sh.at[dbs[b]], ss[b], add=True)

    def wait_scatter(b):
        pltpu.make_async_copy(xbs[b], aggr_sh.at[dbs[b]], ss[b]).wait()

    issue(0, 0)

    def step(s, carry):
        c0 = s * 2

        @pl.when(s > 0)
        def _():
            wait_scatter(1)

        issue(c0 + 1, 1)
        wait_in(c0, 0)
        compute(0)
        scatter(0)

        wait_scatter(0)
        issue(c0 + 2, 0)
        wait_in(c0 + 1, 1)
        compute(1)
        scatter(1)
        return carry

    lax.fori_loop(0, (ncht - 1) // 2, step, 0)

    cl = ncht - 1
    wait_scatter(1)
    wait_in(cl, 0)
    compute(0)
    scatter(0)
    wait_scatter(0)
    plsc.subcore_barrier()

    @pl.when(sid < NS - 1)
    def _():
        pltpu.sync_copy(aggr_sh.at[pl.ds(sid * RB, RB)],
                        out_hbm.at[cid].at[pl.ds(sid * RB, RB)])

    @pl.when(sid == NS - 1)
    def _():
        pltpu.sync_copy(aggr_sh.at[pl.ds((NS - 1) * RB, rem_o)],
                        out_hbm.at[cid].at[pl.ds((NS - 1) * RB, rem_o)])


def _sc_edge_agg(src, dst, ea, x):
    N, D = x.shape
    ncht = src.shape[0] // (NC * NS * CH)
    assert (ncht - 1) % 2 == 0
    zeros = jnp.zeros((N, D), jnp.float32)
    mesh = plsc.VectorSubcoreMesh(
        core_axis_name="c", subcore_axis_name="s", num_cores=NC,
        num_subcores=NS)
    f = pl.kernel(
        functools.partial(_sc_edge_body, N, ncht),
        out_type=jax.ShapeDtypeStruct((NC, N, D), jnp.float32),
        mesh=mesh,
        scratch_types=(
            [pltpu.VMEM((ncht * CH,), jnp.int32)]
            + [pltpu.VMEM((CH,), jnp.int32) for _ in range(2)]
            + [pltpu.VMEM((CH, D), jnp.float32) for _ in range(2)]
            + [pltpu.VMEM((CH // 2, D), jnp.int32) for _ in range(2)]
            + [pltpu.VMEM_SHARED((N, D), jnp.float32)]
            + [pltpu.SemaphoreType.DMA for _ in range(8)]
        ),
    )
    return f(src, dst, ea, x, zeros)


def _mlp_body(x_ref, a0_ref, a1_ref, w1_ref, b1_ref, w2_ref, b2_ref,
              wg1_ref, bg1_ref, wg2_ref, bg2_ref, h_ref, g_ref):
    h0 = x_ref[...] + a0_ref[0] + a1_ref[0]
    t = jnp.maximum(
        jnp.dot(h0, w1_ref[...], preferred_element_type=jnp.float32)
        + b1_ref[...], 0.0)
    h = (jnp.dot(t, w2_ref[...], preferred_element_type=jnp.float32)
         + b2_ref[...])
    h_ref[...] = h
    gt = jnp.maximum(
        jnp.dot(h, wg1_ref[...], preferred_element_type=jnp.float32)
        + bg1_ref[...], 0.0)
    g_ref[...] = (jnp.dot(gt, wg2_ref[...], preferred_element_type=jnp.float32)
                  + bg2_ref[...])


def _node_mlp(x, aggr2, W1, b1, W2, b2, Wg1, bg1, Wg2, bg2):
    N, D = x.shape
    H = W1.shape[1]
    GD = Wg1.shape[1]
    BN = 2000
    grid = (N // BN,)
    return pl.pallas_call(
        _mlp_body,
        grid=grid,
        in_specs=[
            pl.BlockSpec((BN, D), lambda i: (i, 0)),
            pl.BlockSpec((1, BN, D), lambda i: (0, i, 0)),
            pl.BlockSpec((1, BN, D), lambda i: (1, i, 0)),
            pl.BlockSpec((D, H), lambda i: (0, 0)),
            pl.BlockSpec((1, H), lambda i: (0, 0)),
            pl.BlockSpec((H, H), lambda i: (0, 0)),
            pl.BlockSpec((1, H), lambda i: (0, 0)),
            pl.BlockSpec((H, GD), lambda i: (0, 0)),
            pl.BlockSpec((1, GD), lambda i: (0, 0)),
            pl.BlockSpec((GD, 1), lambda i: (0, 0)),
            pl.BlockSpec((1, 1), lambda i: (0, 0)),
        ],
        out_specs=[
            pl.BlockSpec((BN, H), lambda i: (i, 0)),
            pl.BlockSpec((BN, 1), lambda i: (i, 0)),
        ],
        out_shape=[
            jax.ShapeDtypeStruct((N, H), jnp.float32),
            jax.ShapeDtypeStruct((N, 1), jnp.float32),
        ],
    )(x, aggr2, aggr2, W1, b1.reshape(1, H), W2, b2.reshape(1, H),
      Wg1, bg1.reshape(1, GD), Wg2, bg2.reshape(1, 1))


def _pool_body(n_groups, h_ref, g_ref, b_ref, wh_ref, bh_ref,
               logits_ref, pooled_ref):
    g = g_ref[...]
    bid = b_ref[...]
    N = g.shape[0]
    gidx = lax.broadcasted_iota(jnp.int32, (N, n_groups), 1)
    mask = bid == gidx
    gmax = jnp.max(jnp.where(mask, g, -jnp.inf), axis=0, keepdims=True)
    gmax = jnp.where(gmax > -3e38, gmax, 0.0)
    gmax_n = jnp.sum(jnp.where(mask, gmax, 0.0), axis=1, keepdims=True)
    ge = jnp.exp(g - gmax_n)
    gsum = jnp.sum(jnp.where(mask, ge, 0.0), axis=0, keepdims=True)
    gsum_n = jnp.sum(jnp.where(mask, gsum, 0.0), axis=1, keepdims=True)
    alpha = ge / (gsum_n + 1e-16)
    wm = jnp.where(mask, alpha, 0.0)
    pooled = lax.dot_general(wm, h_ref[...], (((0,), (0,)), ((), ())),
                             preferred_element_type=jnp.float32)
    pooled_ref[...] = pooled
    logits_ref[...] = (
        jnp.dot(pooled, wh_ref[...], preferred_element_type=jnp.float32)
        + bh_ref[...])


def _pool_head(h, g, batch, Wh, bh, n_groups):
    N, H = h.shape
    C = Wh.shape[1]
    return pl.pallas_call(
        functools.partial(_pool_body, n_groups),
        out_shape=[
            jax.ShapeDtypeStruct((n_groups, C), jnp.float32),
            jax.ShapeDtypeStruct((n_groups, H), jnp.float32),
        ],
    )(h, g, batch.reshape(N, 1).astype(jnp.int32), Wh, bh.reshape(1, C))


def kernel(x, edge_index, edge_attr, batch, We, be, W1, b1, W2, b2,
           Wg1, bg1, Wg2, bg2, Wh, bh):
    N, D = x.shape
    E, ED = edge_attr.shape
    NW = NC * NS
    EH = E // 2
    CR = CH // 2
    assert E % (CH * NW) == 0
    def reorder(v):
        return jnp.concatenate(
            [v[:EH].reshape(EH // CR, CR), v[EH:].reshape(EH // CR, CR)],
            axis=1).reshape(E)

    src = reorder(edge_index[0])
    dst = reorder(edge_index[1])
    w = jnp.arange(D // 2)
    csel_lo = 32 * (w // 16) + w % 16
    csel_hi = csel_lo + 16
    eap = _edge_proj(edge_attr, We[:, csel_lo], We[:, csel_hi],
                     be[csel_lo], be[csel_hi])
    aggr2 = _sc_edge_agg(src, dst, eap, x)
    h, g = _node_mlp(x, aggr2, W1, b1, W2, b2, Wg1, bg1, Wg2, bg2)
    logits, pooled = _pool_head(h, g, batch, Wh, bh, 64)
    return (logits, pooled)
